# Initial kernel scaffold; baseline (speedup 1.0000x reference)
import jax
import jax.numpy as jnp
from jax.experimental import pallas as pl


def kernel(input, pe, noise):
    raise NotImplementedError("write your pallas kernel here")


# speedup vs baseline: 2.2261x; 2.2261x over previous
import functools
import jax
import jax.numpy as jnp
from jax import lax
from jax.experimental import pallas as pl
from jax.experimental.pallas import tpu as pltpu

PAD_TOKEN = 0


def _rank_gather_body(inp_ref, noise_ref, pe_ref, out_ref):
    inp = inp_ref[...]
    noise = noise_ref[...]
    B, S = inp.shape
    pos = jnp.where(inp == PAD_TOKEN, jnp.inf, noise)
    a = pos[:, :, None]
    b = pos[:, None, :]
    k_i = lax.broadcasted_iota(jnp.int32, (B, S, S), 2)
    j_i = lax.broadcasted_iota(jnp.int32, (B, S, S), 1)
    cmp = (b < a) | ((b == a) & (k_i < j_i))
    rank = jnp.sum(cmp.astype(jnp.float32), axis=2).astype(jnp.int32)
    r_i = lax.broadcasted_iota(jnp.int32, (B, S, S), 2)
    onehot = (rank[:, :, None] == r_i).astype(jnp.float32)
    pe = pe_ref[...]
    for i in range(B):
        out_ref[i] = jnp.dot(onehot[i], pe, preferred_element_type=jnp.float32)


def kernel(input, pe, noise):
    BATCH, SEQ = input.shape
    EMB = pe.shape[1]
    B_BLK = 8
    grid = (BATCH // B_BLK,)
    pe_s = pe[:SEQ]
    return pl.pallas_call(
        _rank_gather_body,
        grid=grid,
        in_specs=[
            pl.BlockSpec((B_BLK, SEQ), lambda i: (i, 0)),
            pl.BlockSpec((B_BLK, SEQ), lambda i: (i, 0)),
            pl.BlockSpec((SEQ, EMB), lambda i: (0, 0)),
        ],
        out_specs=pl.BlockSpec((B_BLK, SEQ, EMB), lambda i: (i, 0, 0)),
        out_shape=jax.ShapeDtypeStruct((BATCH, SEQ, EMB), jnp.float32),
    )(input, noise, pe_s)


# speedup vs baseline: 2.7869x; 1.2519x over previous
import functools
import jax
import jax.numpy as jnp
from jax import lax
from jax.experimental import pallas as pl
from jax.experimental.pallas import tpu as pltpu
from jax.experimental.pallas import tpu_sc as plsc

PAD_TOKEN = 0


def _rank_body(inp_ref, noise_ref, rank_ref):
    inp = inp_ref[...]
    noise = noise_ref[...]
    B, S = inp.shape
    pos = jnp.where(inp == PAD_TOKEN, jnp.inf, noise)
    a = pos[:, :, None]
    b = pos[:, None, :]
    k_i = lax.broadcasted_iota(jnp.int32, (1, S, S), 2)
    j_i = lax.broadcasted_iota(jnp.int32, (1, S, S), 1)
    tie = k_i < j_i
    cmp = (b < a) | ((b == a) & tie)
    rank_ref[...] = jnp.sum(cmp.astype(jnp.float32), axis=2).astype(jnp.int32)


def _compute_ranks(input, noise):
    BATCH, SEQ = input.shape
    B_BLK = 8
    return pl.pallas_call(
        _rank_body,
        grid=(BATCH // B_BLK,),
        in_specs=[
            pl.BlockSpec((B_BLK, SEQ), lambda i: (i, 0)),
            pl.BlockSpec((B_BLK, SEQ), lambda i: (i, 0)),
        ],
        out_specs=pl.BlockSpec((B_BLK, SEQ), lambda i: (i, 0)),
        out_shape=jax.ShapeDtypeStruct((BATCH, SEQ), jnp.int32),
    )(input, noise)


def _pair_table_body(pe_ref, out_ref):
    r1 = pl.program_id(0)
    S, E = pe_ref.shape
    left = jnp.broadcast_to(pe_ref[pl.ds(r1, 1), :], (S, E))
    out_ref[:, :E] = left
    out_ref[:, E:] = pe_ref[...]


def _build_pair_table(pe_s):
    S, E = pe_s.shape
    return pl.pallas_call(
        _pair_table_body,
        grid=(S,),
        in_specs=[pl.BlockSpec((S, E), lambda i: (0, 0))],
        out_specs=pl.BlockSpec((S, 2 * E), lambda i: (i, 0)),
        out_shape=jax.ShapeDtypeStruct((S * S, 2 * E), jnp.float32),
    )(pe_s)


def _make_sc_gather(n_idx, emb2, nc, ns, chunk):
    nw = nc * ns
    assert n_idx % (nw * chunk) == 0
    b_per_w = n_idx // nw
    n_chunks = b_per_w // chunk
    mesh = plsc.VectorSubcoreMesh(core_axis_name="c", subcore_axis_name="s")

    @functools.partial(
        pl.kernel,
        mesh=mesh,
        out_type=jax.ShapeDtypeStruct((n_idx, emb2), jnp.float32),
        scratch_types=[
            pltpu.VMEM((chunk,), jnp.int32),
            pltpu.VMEM((chunk, emb2), jnp.float32),
            pltpu.SemaphoreType.DMA,
        ],
    )
    def sc_gather(idx_hbm, table_hbm, out_hbm, idx_v, rows_v, sem):
        wid = lax.axis_index("s") * nc + lax.axis_index("c")
        base = wid * b_per_w
        for c in range(n_chunks):
            off = base + c * chunk
            pltpu.sync_copy(idx_hbm.at[pl.ds(off, chunk)], idx_v)
            pltpu.async_copy(table_hbm.at[idx_v], rows_v, sem).wait()
            pltpu.sync_copy(rows_v, out_hbm.at[pl.ds(off, chunk)])

    return sc_gather


def kernel(input, pe, noise):
    BATCH, SEQ = input.shape
    EMB = pe.shape[1]
    ranks = _compute_ranks(input, noise)
    r3 = ranks.reshape(BATCH, SEQ // 2, 2)
    pair_idx = (r3[:, :, 0] * SEQ + r3[:, :, 1]).reshape(BATCH * SEQ // 2)
    table = _build_pair_table(pe[:SEQ])
    info = plsc.get_sparse_core_info()
    gather = _make_sc_gather(
        BATCH * SEQ // 2, 2 * EMB, info.num_cores, info.num_subcores, 512
    )
    out = gather(pair_idx, table)
    return out.reshape(BATCH, SEQ, EMB)


# speedup vs baseline: 2.9830x; 1.0704x over previous
import functools
import jax
import jax.numpy as jnp
from jax import lax
from jax.experimental import pallas as pl
from jax.experimental.pallas import tpu as pltpu
from jax.experimental.pallas import tpu_sc as plsc

PAD_TOKEN = 0


def _rank_body(inp_ref, noise_ref, pair_ref):
    inp = inp_ref[...]
    noise = noise_ref[...]
    B, S = inp.shape
    pos = jnp.where(inp == PAD_TOKEN, jnp.inf, noise)
    a = pos[:, :, None]
    b = pos[:, None, :]
    k_i = lax.broadcasted_iota(jnp.int32, (1, S, S), 2)
    j_i = lax.broadcasted_iota(jnp.int32, (1, S, S), 1)
    tie = k_i < j_i
    cmp = (b < a) | ((b == a) & tie)
    rank_f = jnp.sum(cmp.astype(jnp.float32), axis=2)
    kk = lax.broadcasted_iota(jnp.int32, (S, S // 2), 0)
    pp = lax.broadcasted_iota(jnp.int32, (S, S // 2), 1)
    sel = jnp.where(kk == 2 * pp, float(S), 0.0) + jnp.where(
        kk == 2 * pp + 1, 1.0, 0.0
    )
    pair = jnp.dot(rank_f, sel, preferred_element_type=jnp.float32)
    pair_ref[...] = (pair + 0.5).astype(jnp.int32)


def _compute_pair_idx(input, noise):
    BATCH, SEQ = input.shape
    B_BLK = 16
    return pl.pallas_call(
        _rank_body,
        grid=(BATCH // B_BLK,),
        in_specs=[
            pl.BlockSpec((B_BLK, SEQ), lambda i: (i, 0)),
            pl.BlockSpec((B_BLK, SEQ), lambda i: (i, 0)),
        ],
        out_specs=pl.BlockSpec((B_BLK, SEQ // 2), lambda i: (i, 0)),
        out_shape=jax.ShapeDtypeStruct((BATCH, SEQ // 2), jnp.int32),
    )(input, noise)


def _pair_table_body(pe_ref, out_ref):
    r1 = pl.program_id(0)
    S, E = pe_ref.shape
    left = jnp.broadcast_to(pe_ref[pl.ds(r1, 1), :], (S, E))
    out_ref[:, :E] = left
    out_ref[:, E:] = pe_ref[...]


def _build_pair_table(pe_s):
    S, E = pe_s.shape
    return pl.pallas_call(
        _pair_table_body,
        grid=(S,),
        in_specs=[pl.BlockSpec((S, E), lambda i: (0, 0))],
        out_specs=pl.BlockSpec((S, 2 * E), lambda i: (i, 0)),
        out_shape=jax.ShapeDtypeStruct((S * S, 2 * E), jnp.float32),
    )(pe_s)


def _make_sc_gather(n_idx, emb2, nc, ns, chunk):
    nw = nc * ns
    assert n_idx % (nw * chunk) == 0
    b_per_w = n_idx // nw
    n_chunks = b_per_w // chunk
    mesh = plsc.VectorSubcoreMesh(core_axis_name="c", subcore_axis_name="s")

    @functools.partial(
        pl.kernel,
        mesh=mesh,
        out_type=jax.ShapeDtypeStruct((n_idx, emb2), jnp.float32),
        scratch_types=[
            pltpu.VMEM((b_per_w,), jnp.int32),
            pltpu.VMEM((chunk, emb2), jnp.float32),
            pltpu.VMEM((chunk, emb2), jnp.float32),
            pltpu.SemaphoreType.DMA,
            pltpu.SemaphoreType.DMA,
            pltpu.SemaphoreType.DMA,
            pltpu.SemaphoreType.DMA,
        ],
    )
    def sc_gather(idx_hbm, table_hbm, out_hbm, idx_v, rows0, rows1, g0, g1, w0, w1):
        wid = lax.axis_index("s") * nc + lax.axis_index("c")
        base = wid * b_per_w
        pltpu.sync_copy(idx_hbm.at[pl.ds(base, b_per_w)], idx_v)
        rows = [rows0, rows1]
        gsem = [g0, g1]
        wsem = [w0, w1]
        pend_g = [None, None]
        pend_w = [None, None]
        pend_g[0] = pltpu.async_copy(
            table_hbm.at[idx_v.at[pl.ds(0, chunk)]], rows[0], gsem[0]
        )
        for c in range(n_chunks):
            i = c % 2
            ni = (c + 1) % 2
            if c + 1 < n_chunks:
                if pend_w[ni] is not None:
                    pend_w[ni].wait()
                    pend_w[ni] = None
                pend_g[ni] = pltpu.async_copy(
                    table_hbm.at[idx_v.at[pl.ds((c + 1) * chunk, chunk)]],
                    rows[ni],
                    gsem[ni],
                )
            pend_g[i].wait()
            pend_g[i] = None
            pend_w[i] = pltpu.async_copy(
                rows[i], out_hbm.at[pl.ds(base + c * chunk, chunk)], wsem[i]
            )
        for i in range(2):
            if pend_w[i] is not None:
                pend_w[i].wait()

    return sc_gather


def kernel(input, pe, noise):
    BATCH, SEQ = input.shape
    EMB = pe.shape[1]
    pair_idx = _compute_pair_idx(input, noise).reshape(BATCH * SEQ // 2)
    table = _build_pair_table(pe[:SEQ])
    info = plsc.get_sparse_core_info()
    gather = _make_sc_gather(
        BATCH * SEQ // 2, 2 * EMB, info.num_cores, info.num_subcores, 320
    )
    out = gather(pair_idx, table)
    return out.reshape(BATCH, SEQ, EMB)
